# Initial kernel scaffold; baseline (speedup 1.0000x reference)
import jax
import jax.numpy as jnp
from jax.experimental import pallas as pl


def kernel(atom_type, pos, bond_index, bond_type, batch, time_step, edge_index, edge_type, edge_length, params):
    raise NotImplementedError("write your pallas kernel here")


# speedup vs baseline: 1.3060x; 1.3060x over previous
import functools
import math

import jax
import jax.numpy as jnp
from jax import lax
from jax.experimental import pallas as pl
from jax.experimental.pallas import tpu as pltpu
from jax.experimental.pallas import tpu_sc as plsc

N = 10000
E = 160000
H = 128
NCONV = 6
NCONV_L = 4
CUTOFF = 10.0

NSC = 2
NTILE = 16
NW = NSC * NTILE
E_PAD = 163840
EPT = E_PAD // NW
CH = 128
NCHUNK = EPT // CH
N_PAD = 10240
NPT = N_PAD // NTILE
LN2 = math.log(2.0)


def _ssp(x):
    return jnp.maximum(x, 0.0) + jnp.log1p(jnp.exp(-jnp.abs(x))) - LN2


_BE = 2048


def _t0_body(el_r, et_r, eW1_r, eb1_r, eW2_r, eb2_r, bemb_r,
             sW1_r, sb1_r, sW2_r, sb2_r,
             ea_o, eap_o, mask_o, wf_o):
    el = el_r[...]
    et = et_r[...]
    d1 = jnp.maximum(el * eW1_r[...] + eb1_r[...], 0.0)
    d2 = jnp.dot(d1, eW2_r[...], preferred_element_type=jnp.float32) + eb2_r[...]
    oh = (lax.broadcasted_iota(jnp.int32, (_BE, 128), 1) == et).astype(jnp.float32)
    bond = jnp.dot(oh, bemb_r[...], preferred_element_type=jnp.float32, precision=lax.Precision.HIGHEST)
    ea = d2 * bond
    ea_o[...] = ea
    mask = (et > 0).astype(jnp.float32)
    mask_o[...] = mask
    eap_o[...] = jnp.where(mask > 0.5, ea, -1e30)
    C = 0.5 * (jnp.cos(el * (math.pi / CUTOFF)) + 1.0)
    C = C * (el <= CUTOFF).astype(jnp.float32)
    for i in range(NCONV):
        t = _ssp(jnp.dot(ea, sW1_r[i], preferred_element_type=jnp.float32) + sb1_r[i])
        w = jnp.dot(t, sW2_r[i], preferred_element_type=jnp.float32) + sb2_r[i]
        wf_o[i] = w * C


def _edge_precompute(el_p, et_p, p):
    grid = (E_PAD // _BE,)
    eb = lambda: pl.BlockSpec((_BE, 1), lambda e: (e, 0))
    full = lambda *s: pl.BlockSpec(s, lambda e: tuple(0 for _ in s))
    bemb = jnp.pad(p['bond_emb'], ((0, 28), (0, 0)))
    return pl.pallas_call(
        _t0_body,
        grid=grid,
        in_specs=[
            eb(), eb(),
            full(1, H), full(1, H), full(H, H), full(1, H), full(128, H),
            full(NCONV, H, H), full(NCONV, 1, H), full(NCONV, H, H), full(NCONV, 1, H),
        ],
        out_specs=[
            pl.BlockSpec((_BE, H), lambda e: (e, 0)),
            pl.BlockSpec((_BE, H), lambda e: (e, 0)),
            pl.BlockSpec((_BE, 1), lambda e: (e, 0)),
            pl.BlockSpec((NCONV, _BE, H), lambda e: (0, e, 0)),
        ],
        out_shape=[
            jax.ShapeDtypeStruct((E_PAD, H), jnp.float32),
            jax.ShapeDtypeStruct((E_PAD, H), jnp.float32),
            jax.ShapeDtypeStruct((E_PAD, 1), jnp.float32),
            jax.ShapeDtypeStruct((NCONV, E_PAD, H), jnp.float32),
        ],
    )(el_p, et_p,
      p['ee_W1'], p['ee_b1'].reshape(1, H), p['ee_W2'], p['ee_b2'].reshape(1, H),
      bemb,
      p['sch_eW1'], p['sch_eb1'].reshape(NCONV, 1, H),
      p['sch_eW2'], p['sch_eb2'].reshape(NCONV, 1, H))


def _tinit_body(at_r, semb_r, gemb_r, lin1_r, h_o, hg_o, y_o):
    at = at_r[...]
    oh = (lax.broadcasted_iota(jnp.int32, (2000, 128), 1) == at).astype(jnp.float32)
    h = jnp.dot(oh, semb_r[...], preferred_element_type=jnp.float32, precision=lax.Precision.HIGHEST)
    h_o[...] = h
    hg_o[...] = jnp.dot(oh, gemb_r[...], preferred_element_type=jnp.float32, precision=lax.Precision.HIGHEST)
    y_o[...] = jnp.dot(h, lin1_r[...], preferred_element_type=jnp.float32)


def _node_init(at2, p):
    semb = jnp.pad(p['sch_emb'], ((0, 28), (0, 0)))
    gemb = jnp.pad(p['gin_emb'], ((0, 28), (0, 0)))
    full = lambda *s: pl.BlockSpec(s, lambda e: tuple(0 for _ in s))
    nblk = lambda w: pl.BlockSpec((2000, w), lambda e: (e, 0))
    return pl.pallas_call(
        _tinit_body,
        grid=(N // 2000,),
        in_specs=[nblk(1), full(128, H), full(128, H), full(H, H)],
        out_specs=[nblk(H), nblk(H), nblk(H)],
        out_shape=[jax.ShapeDtypeStruct((N, H), jnp.float32)] * 3,
    )(at2, semb, gemb, p['sch_lin1'][0])


def _sch_update_body(p_r, h_r, W2_r, b2_r, W3_r, b3_r, W1n_r, h_o, y_o):
    agg = p_r[:N] + p_r[N_PAD:N_PAD + N]
    m = _ssp(jnp.dot(agg, W2_r[...], preferred_element_type=jnp.float32) + b2_r[...])
    m = jnp.dot(m, W3_r[...], preferred_element_type=jnp.float32) + b3_r[...]
    h = h_r[...] + m
    h_o[...] = h
    y_o[...] = jnp.dot(h, W1n_r[...], preferred_element_type=jnp.float32)


def _sch_update(partials, h, W2, b2, W3, b3, W1n):
    return pl.pallas_call(
        _sch_update_body,
        out_shape=[jax.ShapeDtypeStruct((N, H), jnp.float32)] * 2,
    )(partials, h, W2, b2.reshape(1, H), W3, b3.reshape(1, H), W1n)


def _gin_update_body(relu_out, p_r, hg_r, eps_r, W1_r, b1_r, W2_r, b2_r, hg_o):
    agg = p_r[:N] + p_r[N_PAD:N_PAD + N]
    out = (1.0 + eps_r[...]) * hg_r[...] + agg
    a = jnp.maximum(jnp.dot(out, W1_r[...], preferred_element_type=jnp.float32) + b1_r[...], 0.0)
    o2 = jnp.dot(a, W2_r[...], preferred_element_type=jnp.float32) + b2_r[...]
    hg_o[...] = jnp.maximum(o2, 0.0) if relu_out else o2


def _gin_update(partials, hg, eps, W1, b1, W2, b2, relu_out):
    return pl.pallas_call(
        functools.partial(_gin_update_body, relu_out),
        out_shape=jax.ShapeDtypeStruct((N, H), jnp.float32),
    )(partials, hg, eps.reshape(1, 1), W1, b1.reshape(1, H), W2, b2.reshape(1, H))


def _pair_body(use_mask, prod_r, ea_r, W1a_r, W1b_r, b1_r, W2_r, b2_r, W3_r, b3_r, mask_r, o_o):
    eg = jnp.dot(prod_r[...], W1a_r[...], preferred_element_type=jnp.float32)
    eg = eg + jnp.dot(ea_r[...], W1b_r[...], preferred_element_type=jnp.float32)
    eg = jnp.maximum(eg + b1_r[...], 0.0)
    eg = jnp.maximum(jnp.dot(eg, W2_r[...], preferred_element_type=jnp.float32) + b2_r[...], 0.0)
    o = jnp.dot(eg, W3_r[...], preferred_element_type=jnp.float32) + b3_r[...]
    if use_mask:
        o = o * mask_r[...]
    o_o[...] = o


def _pair_mlp(prod, ea, W1, b1, W2, b2, W3, b3, maskf, use_mask):
    full = lambda *s: pl.BlockSpec(s, lambda e: tuple(0 for _ in s))
    return pl.pallas_call(
        functools.partial(_pair_body, use_mask),
        grid=(E_PAD // _BE,),
        in_specs=[
            pl.BlockSpec((_BE, H), lambda e: (e, 0)),
            pl.BlockSpec((_BE, H), lambda e: (e, 0)),
            full(H, H), full(H, H), full(1, H),
            full(H, H // 2), full(1, H // 2), full(H // 2, 1), full(1, 1),
            pl.BlockSpec((_BE, 1), lambda e: (e, 0)),
        ],
        out_specs=pl.BlockSpec((_BE, 1), lambda e: (e, 0)),
        out_shape=jax.ShapeDtypeStruct((E_PAD, 1), jnp.float32),
    )(prod, ea, W1[:H], W1[H:], b1.reshape(1, H), W2, b2.reshape(1, H // 2),
      W3, b3.reshape(1, 1), maskf)


def _vec_combine(rows, feat, op):
    def grp(g, _):
        for rr in range(8):
            row = g * 8 + rr
            for j in range(8):
                sl = (row, pl.ds(j * 16, 16))
                a = rows[sl]
                b = feat[sl]
                if op == 'mul':
                    r = a * b
                else:
                    r = jnp.maximum(a + b, 0.0)
                rows[sl] = r
        return 0

    lax.fori_loop(0, CH // 8, grp, 0)


def _sc_msg(table, edgefeat, src_p, dst_p, zeros_n, op):
    mesh = plsc.VectorSubcoreMesh(core_axis_name="c", subcore_axis_name="s",
                                  num_cores=NSC, num_subcores=NTILE)

    @functools.partial(
        pl.kernel,
        out_type=jax.ShapeDtypeStruct((2 * N_PAD, H), jnp.float32),
        mesh=mesh,
        scratch_types=[
            pltpu.VMEM((CH,), jnp.int32),
            pltpu.VMEM((CH,), jnp.int32),
            pltpu.VMEM((CH, H), jnp.float32),
            pltpu.VMEM((CH, H), jnp.float32),
            pltpu.VMEM_SHARED((N_PAD, H), jnp.float32),
            pltpu.SemaphoreType.DMA,
        ],
    )
    def k(tab_h, ef_h, src_h, dst_h, z_h, out_h, sidx, didx, rows, feat, agg, sem):
        c = lax.axis_index("c")
        s = lax.axis_index("s")
        wid = c * NTILE + s
        pltpu.sync_copy(z_h, agg.at[pl.ds(s * NPT, NPT)])
        plsc.subcore_barrier()

        def chunk(kk, _):
            base = wid * EPT + kk * CH
            pltpu.sync_copy(src_h.at[pl.ds(base, CH)], sidx)
            pltpu.async_copy(tab_h.at[sidx], rows, sem).wait()
            pltpu.sync_copy(ef_h.at[pl.ds(base, CH)], feat)
            _vec_combine(rows, feat, op)
            pltpu.sync_copy(dst_h.at[pl.ds(base, CH)], didx)
            pltpu.sync_copy(rows, agg.at[didx], add=True)
            return 0

        lax.fori_loop(0, NCHUNK, chunk, 0)
        plsc.subcore_barrier()
        for t in range(5):
            off = s * NPT + t * 128
            pltpu.sync_copy(agg.at[pl.ds(off, 128)],
                            out_h.at[pl.ds(c * N_PAD + off, 128)])

    return k(table, edgefeat, src_p, dst_p, zeros_n)


def _sc_pair_prod(hsrc_tab, src_p, dst_p):
    mesh = plsc.VectorSubcoreMesh(core_axis_name="c", subcore_axis_name="s",
                                  num_cores=NSC, num_subcores=NTILE)

    @functools.partial(
        pl.kernel,
        out_type=jax.ShapeDtypeStruct((E_PAD, H), jnp.float32),
        mesh=mesh,
        scratch_types=[
            pltpu.VMEM((CH,), jnp.int32),
            pltpu.VMEM((CH,), jnp.int32),
            pltpu.VMEM((CH, H), jnp.float32),
            pltpu.VMEM((CH, H), jnp.float32),
            pltpu.SemaphoreType.DMA,
            pltpu.SemaphoreType.DMA,
        ],
    )
    def k(tab_h, src_h, dst_h, out_h, sidx, didx, rows, rows2, sem, sem2):
        c = lax.axis_index("c")
        s = lax.axis_index("s")
        wid = c * NTILE + s

        def chunk(kk, _):
            base = wid * EPT + kk * CH
            pltpu.sync_copy(src_h.at[pl.ds(base, CH)], sidx)
            pltpu.async_copy(tab_h.at[sidx], rows, sem).wait()
            pltpu.sync_copy(dst_h.at[pl.ds(base, CH)], didx)
            pltpu.async_copy(tab_h.at[didx], rows2, sem2).wait()
            _vec_combine(rows, rows2, 'mul')
            pltpu.sync_copy(rows, out_h.at[pl.ds(base, CH)])
            return 0

        lax.fori_loop(0, NCHUNK, chunk, 0)

    return k(hsrc_tab, src_p, dst_p)


def kernel(atom_type, pos, bond_index, bond_type, batch, time_step,
           edge_index, edge_type, edge_length, params):
    p = params
    src = edge_index[0].astype(jnp.int32)
    dst = edge_index[1].astype(jnp.int32)
    et = edge_type.astype(jnp.int32)
    pad = E_PAD - E
    src_p = jnp.pad(src, (0, pad))
    dst_p = jnp.pad(dst, (0, pad))
    et_p = jnp.pad(et, (0, pad), constant_values=-1).reshape(E_PAD, 1)
    el_p = jnp.pad(edge_length, ((0, pad), (0, 0)),
                   constant_values=CUTOFF + 1.0)
    at2 = atom_type.astype(jnp.int32).reshape(N, 1)
    zeros_n = jnp.zeros((NPT, H), jnp.float32)

    ea, eap, maskf, wf = _edge_precompute(el_p, et_p, p)
    h, hg, y = _node_init(at2, p)

    for i in range(NCONV):
        partials = _sc_msg(y, wf[i], src_p, dst_p, zeros_n, 'mul')
        W1n = p['sch_lin1'][(i + 1) % NCONV]
        h, y = _sch_update(partials, h,
                           p['sch_lin2'][i], p['sch_lin2b'][i],
                           p['sch_lin3'][i], p['sch_lin3b'][i], W1n)

    for i in range(NCONV_L):
        partials = _sc_msg(hg, eap, src_p, dst_p, zeros_n, 'addrelu')
        hg = _gin_update(partials, hg, p['gin_eps'][i],
                         p['gin_W1'][i], p['gin_b1'][i],
                         p['gin_W2'][i], p['gin_b2'][i],
                         relu_out=(i < NCONV_L - 1))

    prod_g = _sc_pair_prod(h, src_p, dst_p)
    out_g = _pair_mlp(prod_g, ea, p['gg_W1'], p['gg_b1'], p['gg_W2'],
                      p['gg_b2'], p['gg_W3'], p['gg_b3'], maskf, False)
    prod_l = _sc_pair_prod(hg, src_p, dst_p)
    out_l = _pair_mlp(prod_l, ea, p['gl_W1'], p['gl_b1'], p['gl_W2'],
                      p['gl_b2'], p['gl_W3'], p['gl_b3'], maskf, True)

    return out_g[:E], out_l[:E]


# speedup vs baseline: 1.4492x; 1.1096x over previous
import functools
import math

import jax
import jax.numpy as jnp
from jax import lax
from jax.experimental import pallas as pl
from jax.experimental.pallas import tpu as pltpu
from jax.experimental.pallas import tpu_sc as plsc

N = 10000
E = 160000
H = 128
NCONV = 6
NCONV_L = 4
CUTOFF = 10.0

NSC = 2
NTILE = 16
NW = NSC * NTILE
E_PAD = 163840
EPT = E_PAD // NW
CH = 128
NCHUNK = EPT // CH
N_PAD = 10240
NPT = N_PAD // NTILE
LN2 = math.log(2.0)


def _ssp(x):
    return jnp.maximum(x, 0.0) + jnp.log1p(jnp.exp(-jnp.abs(x))) - LN2


_BE = 2048


def _t0_body(el_r, et_r, eW1_r, eb1_r, eW2_r, eb2_r, bemb_r,
             sW1_r, sb1_r, sW2_r, sb2_r,
             ea_o, eap_o, mask_o, wf_o):
    el = el_r[...]
    et = et_r[...]
    d1 = jnp.maximum(el * eW1_r[...] + eb1_r[...], 0.0)
    d2 = jnp.dot(d1, eW2_r[...], preferred_element_type=jnp.float32) + eb2_r[...]
    oh = (lax.broadcasted_iota(jnp.int32, (_BE, 128), 1) == et).astype(jnp.float32)
    bond = jnp.dot(oh, bemb_r[...], preferred_element_type=jnp.float32, precision=lax.Precision.HIGHEST)
    ea = d2 * bond
    ea_o[...] = ea
    mask = (et > 0).astype(jnp.float32)
    mask_o[...] = mask
    eap_o[...] = jnp.where(mask > 0.5, ea, -1e30)
    C = 0.5 * (jnp.cos(el * (math.pi / CUTOFF)) + 1.0)
    C = C * (el <= CUTOFF).astype(jnp.float32)
    for i in range(NCONV):
        t = _ssp(jnp.dot(ea, sW1_r[i], preferred_element_type=jnp.float32) + sb1_r[i])
        w = jnp.dot(t, sW2_r[i], preferred_element_type=jnp.float32) + sb2_r[i]
        wf_o[i] = w * C


def _edge_precompute(el_p, et_p, p):
    grid = (E_PAD // _BE,)
    eb = lambda: pl.BlockSpec((_BE, 1), lambda e: (e, 0))
    full = lambda *s: pl.BlockSpec(s, lambda e: tuple(0 for _ in s))
    bemb = jnp.pad(p['bond_emb'], ((0, 28), (0, 0)))
    return pl.pallas_call(
        _t0_body,
        grid=grid,
        in_specs=[
            eb(), eb(),
            full(1, H), full(1, H), full(H, H), full(1, H), full(128, H),
            full(NCONV, H, H), full(NCONV, 1, H), full(NCONV, H, H), full(NCONV, 1, H),
        ],
        out_specs=[
            pl.BlockSpec((_BE, H), lambda e: (e, 0)),
            pl.BlockSpec((_BE, H), lambda e: (e, 0)),
            pl.BlockSpec((_BE, 1), lambda e: (e, 0)),
            pl.BlockSpec((NCONV, _BE, H), lambda e: (0, e, 0)),
        ],
        out_shape=[
            jax.ShapeDtypeStruct((E_PAD, H), jnp.float32),
            jax.ShapeDtypeStruct((E_PAD, H), jnp.float32),
            jax.ShapeDtypeStruct((E_PAD, 1), jnp.float32),
            jax.ShapeDtypeStruct((NCONV, E_PAD, H), jnp.float32),
        ],
    )(el_p, et_p,
      p['ee_W1'], p['ee_b1'].reshape(1, H), p['ee_W2'], p['ee_b2'].reshape(1, H),
      bemb,
      p['sch_eW1'], p['sch_eb1'].reshape(NCONV, 1, H),
      p['sch_eW2'], p['sch_eb2'].reshape(NCONV, 1, H))


def _tinit_body(at_r, semb_r, gemb_r, lin1_r, h_o, hg_o, y_o):
    at = at_r[...]
    oh = (lax.broadcasted_iota(jnp.int32, (2000, 128), 1) == at).astype(jnp.float32)
    h = jnp.dot(oh, semb_r[...], preferred_element_type=jnp.float32, precision=lax.Precision.HIGHEST)
    h_o[...] = h
    hg_o[...] = jnp.dot(oh, gemb_r[...], preferred_element_type=jnp.float32, precision=lax.Precision.HIGHEST)
    y_o[...] = jnp.dot(h, lin1_r[...], preferred_element_type=jnp.float32)


def _node_init(at2, p):
    semb = jnp.pad(p['sch_emb'], ((0, 28), (0, 0)))
    gemb = jnp.pad(p['gin_emb'], ((0, 28), (0, 0)))
    full = lambda *s: pl.BlockSpec(s, lambda e: tuple(0 for _ in s))
    nblk = lambda w: pl.BlockSpec((2000, w), lambda e: (e, 0))
    return pl.pallas_call(
        _tinit_body,
        grid=(N // 2000,),
        in_specs=[nblk(1), full(128, H), full(128, H), full(H, H)],
        out_specs=[nblk(H), nblk(H), nblk(H)],
        out_shape=[jax.ShapeDtypeStruct((N, H), jnp.float32)] * 3,
    )(at2, semb, gemb, p['sch_lin1'][0])


def _sch_update_body(p_r, h_r, W2_r, b2_r, W3_r, b3_r, W1n_r, h_o, y_o):
    agg = p_r[:N] + p_r[N_PAD:N_PAD + N]
    m = _ssp(jnp.dot(agg, W2_r[...], preferred_element_type=jnp.float32) + b2_r[...])
    m = jnp.dot(m, W3_r[...], preferred_element_type=jnp.float32) + b3_r[...]
    h = h_r[...] + m
    h_o[...] = h
    y_o[...] = jnp.dot(h, W1n_r[...], preferred_element_type=jnp.float32)


def _sch_update(partials, h, W2, b2, W3, b3, W1n):
    return pl.pallas_call(
        _sch_update_body,
        out_shape=[jax.ShapeDtypeStruct((N, H), jnp.float32)] * 2,
    )(partials, h, W2, b2.reshape(1, H), W3, b3.reshape(1, H), W1n)


def _gin_update_body(relu_out, p_r, hg_r, eps_r, W1_r, b1_r, W2_r, b2_r, hg_o):
    agg = p_r[:N] + p_r[N_PAD:N_PAD + N]
    out = (1.0 + eps_r[...]) * hg_r[...] + agg
    a = jnp.maximum(jnp.dot(out, W1_r[...], preferred_element_type=jnp.float32) + b1_r[...], 0.0)
    o2 = jnp.dot(a, W2_r[...], preferred_element_type=jnp.float32) + b2_r[...]
    hg_o[...] = jnp.maximum(o2, 0.0) if relu_out else o2


def _gin_update(partials, hg, eps, W1, b1, W2, b2, relu_out):
    return pl.pallas_call(
        functools.partial(_gin_update_body, relu_out),
        out_shape=jax.ShapeDtypeStruct((N, H), jnp.float32),
    )(partials, hg, eps.reshape(1, 1), W1, b1.reshape(1, H), W2, b2.reshape(1, H))


def _pair_body(use_mask, prod_r, ea_r, W1a_r, W1b_r, b1_r, W2_r, b2_r, W3_r, b3_r, mask_r, o_o):
    eg = jnp.dot(prod_r[...], W1a_r[...], preferred_element_type=jnp.float32)
    eg = eg + jnp.dot(ea_r[...], W1b_r[...], preferred_element_type=jnp.float32)
    eg = jnp.maximum(eg + b1_r[...], 0.0)
    eg = jnp.maximum(jnp.dot(eg, W2_r[...], preferred_element_type=jnp.float32) + b2_r[...], 0.0)
    o = jnp.dot(eg, W3_r[...], preferred_element_type=jnp.float32) + b3_r[...]
    if use_mask:
        o = o * mask_r[...]
    o_o[...] = o


def _pair_mlp(prod, ea, W1, b1, W2, b2, W3, b3, maskf, use_mask):
    full = lambda *s: pl.BlockSpec(s, lambda e: tuple(0 for _ in s))
    return pl.pallas_call(
        functools.partial(_pair_body, use_mask),
        grid=(E_PAD // _BE,),
        in_specs=[
            pl.BlockSpec((_BE, H), lambda e: (e, 0)),
            pl.BlockSpec((_BE, H), lambda e: (e, 0)),
            full(H, H), full(H, H), full(1, H),
            full(H, H // 2), full(1, H // 2), full(H // 2, 1), full(1, 1),
            pl.BlockSpec((_BE, 1), lambda e: (e, 0)),
        ],
        out_specs=pl.BlockSpec((_BE, 1), lambda e: (e, 0)),
        out_shape=jax.ShapeDtypeStruct((E_PAD, 1), jnp.float32),
    )(prod, ea, W1[:H], W1[H:], b1.reshape(1, H), W2, b2.reshape(1, H // 2),
      W3, b3.reshape(1, 1), maskf)


def _vec_combine(rows, feat, outb, op):
    def grp(g, _):
        for rr in range(8):
            row = g * 8 + rr
            for j in range(8):
                sl = (row, pl.ds(j * 16, 16))
                a = rows[sl]
                b = feat[sl]
                outb[sl] = a * b if op == 'mul' else jnp.maximum(a + b, 0.0)
        return 0

    lax.fori_loop(0, CH // 8, grp, 0)


def _sc_msg(table, edgefeat, src3, dst3, zeros_n, op):
    mesh = plsc.VectorSubcoreMesh(core_axis_name="c", subcore_axis_name="s",
                                  num_cores=NSC, num_subcores=NTILE)

    @functools.partial(
        pl.kernel,
        out_type=jax.ShapeDtypeStruct((2 * N_PAD, H), jnp.float32),
        mesh=mesh,
        scratch_types=[
            pltpu.VMEM((NCHUNK, CH), jnp.int32),
            pltpu.VMEM((NCHUNK, CH), jnp.int32),
            pltpu.VMEM((CH, H), jnp.float32),
            pltpu.VMEM((CH, H), jnp.float32),
            pltpu.VMEM_SHARED((N_PAD, H), jnp.float32),
            pltpu.SemaphoreType.DMA,
            pltpu.SemaphoreType.DMA,
        ],
    )
    def k(tab_h, ef_h, src_h, dst_h, z_h, out_h,
          sidx_all, didx_all, rows, feat, agg, gsem, ssem):
        c = lax.axis_index("c")
        s = lax.axis_index("s")
        wid = c * NTILE + s
        pltpu.sync_copy(z_h, agg.at[pl.ds(s * NPT, NPT)])
        pltpu.sync_copy(src_h.at[wid], sidx_all)
        pltpu.sync_copy(dst_h.at[wid], didx_all)
        plsc.subcore_barrier()

        def wait_chunk(ref, sem):
            pltpu.make_async_copy(ef_h.at[pl.ds(0, CH)], ref, sem).wait()

        pltpu.async_copy(tab_h.at[sidx_all.at[0]], rows, gsem)

        def body(kk, _):
            wait_chunk(rows, gsem)

            @pl.when(kk > 0)
            def _():
                wait_chunk(feat, ssem)

            pltpu.sync_copy(ef_h.at[pl.ds((wid * NCHUNK + kk) * CH, CH)], feat)
            _vec_combine(rows, feat, feat, op)

            @pl.when(kk < NCHUNK - 1)
            def _():
                pltpu.async_copy(tab_h.at[sidx_all.at[kk + 1]], rows, gsem)

            pltpu.async_copy(feat, agg.at[didx_all.at[kk]], ssem, add=True)
            return 0

        lax.fori_loop(0, NCHUNK, body, 0)
        wait_chunk(feat, ssem)
        plsc.subcore_barrier()
        for t in range(5):
            off = s * NPT + t * 128
            pltpu.sync_copy(agg.at[pl.ds(off, 128)],
                            out_h.at[pl.ds(c * N_PAD + off, 128)])

    return k(table, edgefeat, src3, dst3, zeros_n)


def _sc_pair_prod(hsrc_tab, src3, dst3):
    mesh = plsc.VectorSubcoreMesh(core_axis_name="c", subcore_axis_name="s",
                                  num_cores=NSC, num_subcores=NTILE)

    @functools.partial(
        pl.kernel,
        out_type=jax.ShapeDtypeStruct((E_PAD, H), jnp.float32),
        mesh=mesh,
        scratch_types=[
            pltpu.VMEM((NCHUNK, CH), jnp.int32),
            pltpu.VMEM((NCHUNK, CH), jnp.int32),
            [pltpu.VMEM((CH, H), jnp.float32)] * 2,
            [pltpu.VMEM((CH, H), jnp.float32)] * 2,
            [pltpu.SemaphoreType.DMA] * 2,
            [pltpu.SemaphoreType.DMA] * 2,
            [pltpu.SemaphoreType.DMA] * 2,
        ],
    )
    def k(tab_h, src_h, dst_h, out_h,
          sidx_all, didx_all, ra, rb, gsa, gsb, ssem):
        c = lax.axis_index("c")
        s = lax.axis_index("s")
        wid = c * NTILE + s
        pltpu.sync_copy(src_h.at[wid], sidx_all)
        pltpu.sync_copy(dst_h.at[wid], didx_all)

        def wait_chunk(ref, sem):
            pltpu.make_async_copy(tab_h.at[pl.ds(0, CH)], ref, sem).wait()

        def start(cc, b):
            pltpu.async_copy(tab_h.at[sidx_all.at[cc]], ra[b], gsa[b])
            pltpu.async_copy(tab_h.at[didx_all.at[cc]], rb[b], gsb[b])

        start(0, 0)

        def half(kk, b):
            wait_chunk(ra[b], gsa[b])
            wait_chunk(rb[b], gsb[b])
            _vec_combine(ra[b], rb[b], ra[b], 'mul')

            @pl.when(kk + 1 < NCHUNK)
            def _():
                @pl.when(kk >= 1)
                def _():
                    wait_chunk(ra[1 - b], ssem[1 - b])
                start(kk + 1, 1 - b)

            pltpu.async_copy(
                ra[b], out_h.at[pl.ds((wid * NCHUNK + kk) * CH, CH)], ssem[b])

        def body(i, _):
            half(2 * i, 0)
            half(2 * i + 1, 1)
            return 0

        lax.fori_loop(0, NCHUNK // 2, body, 0)
        wait_chunk(ra[0], ssem[0])
        wait_chunk(ra[1], ssem[1])

    return k(hsrc_tab, src3, dst3)


def kernel(atom_type, pos, bond_index, bond_type, batch, time_step,
           edge_index, edge_type, edge_length, params):
    p = params
    src = edge_index[0].astype(jnp.int32)
    dst = edge_index[1].astype(jnp.int32)
    et = edge_type.astype(jnp.int32)
    pad = E_PAD - E
    src3 = jnp.pad(src, (0, pad)).reshape(NW, NCHUNK, CH)
    dst3 = jnp.pad(dst, (0, pad)).reshape(NW, NCHUNK, CH)
    et_p = jnp.pad(et, (0, pad), constant_values=-1).reshape(E_PAD, 1)
    el_p = jnp.pad(edge_length, ((0, pad), (0, 0)),
                   constant_values=CUTOFF + 1.0)
    at2 = atom_type.astype(jnp.int32).reshape(N, 1)
    zeros_n = jnp.zeros((NPT, H), jnp.float32)

    ea, eap, maskf, wf = _edge_precompute(el_p, et_p, p)
    h, hg, y = _node_init(at2, p)

    for i in range(NCONV):
        partials = _sc_msg(y, wf[i], src3, dst3, zeros_n, 'mul')
        W1n = p['sch_lin1'][(i + 1) % NCONV]
        h, y = _sch_update(partials, h,
                           p['sch_lin2'][i], p['sch_lin2b'][i],
                           p['sch_lin3'][i], p['sch_lin3b'][i], W1n)

    for i in range(NCONV_L):
        partials = _sc_msg(hg, eap, src3, dst3, zeros_n, 'addrelu')
        hg = _gin_update(partials, hg, p['gin_eps'][i],
                         p['gin_W1'][i], p['gin_b1'][i],
                         p['gin_W2'][i], p['gin_b2'][i],
                         relu_out=(i < NCONV_L - 1))

    prod_g = _sc_pair_prod(h, src3, dst3)
    out_g = _pair_mlp(prod_g, ea, p['gg_W1'], p['gg_b1'], p['gg_W2'],
                      p['gg_b2'], p['gg_W3'], p['gg_b3'], maskf, False)
    prod_l = _sc_pair_prod(hg, src3, dst3)
    out_l = _pair_mlp(prod_l, ea, p['gl_W1'], p['gl_b1'], p['gl_W2'],
                      p['gl_b2'], p['gl_W3'], p['gl_b3'], maskf, True)

    return out_g[:E], out_l[:E]
